# baseline (speedup 1.0000x reference)
import functools
import jax
import jax.numpy as jnp
from jax.experimental import pallas as pl
from jax.experimental.pallas import tpu as pltpu

N = 10000
K = 30
P = 2

_RB_SEL = 80
_RB_ASM = 80
_RB_MM = 400

_INT_MIN = -(2 ** 31)


def _f32_key(s):
    b = jax.lax.bitcast_convert_type(s, jnp.int32)
    return jnp.where(b >= 0, b, (~b) ^ _INT_MIN)


def _key_f32(k):
    b = jnp.where(k >= 0, k, ~(k ^ _INT_MIN))
    return jax.lax.bitcast_convert_type(b, jnp.float32)


def _select_kernel(s_ref, ov_ref, oc_ref, key_ref):
    key_ref[...] = _f32_key(s_ref[...])
    key = key_ref[...]
    lo = jnp.min(key, axis=1, keepdims=True)
    hi = jnp.max(key, axis=1, keepdims=True) + 1

    def body(_, carry):
        def active(c):
            lo, hi = c
            mid = (lo & hi) + ((lo ^ hi) >> 1)
            cnt = jnp.sum((key >= mid).astype(jnp.float32), axis=1,
                          keepdims=True)
            pred = cnt >= K
            return jnp.where(pred, mid, lo), jnp.where(pred, hi, mid)

        lo, hi = carry
        return jax.lax.cond(jnp.any(hi > lo + 1), active, lambda c: c,
                            (lo, hi))

    lo, hi = jax.lax.fori_loop(0, 32, body, (lo, hi))
    k30 = lo

    m = K - jnp.sum((key > k30).astype(jnp.float32), axis=1, keepdims=True)
    eq = (key == k30).astype(jnp.float32)
    col = jax.lax.broadcasted_iota(jnp.int32, key.shape, 1)

    def body2(_, carry):
        lo2, hi2 = carry
        mid = (lo2 + hi2) >> 1
        cnt = jnp.sum(eq * (col <= mid).astype(jnp.float32), axis=1,
                      keepdims=True)
        pred = cnt >= m
        return jnp.where(pred, lo2, mid + 1), jnp.where(pred, mid, hi2)

    zero = jnp.zeros_like(k30)
    lo2, hi2 = jax.lax.fori_loop(0, 14, body2, (zero, zero + (N - 1)))

    ov_ref[...] = _key_f32(k30)
    oc_ref[...] = lo2


def _select(s):
    return pl.pallas_call(
        _select_kernel,
        grid=(N // _RB_SEL,),
        in_specs=[pl.BlockSpec((_RB_SEL, N), lambda i: (i, 0))],
        out_specs=[
            pl.BlockSpec((_RB_SEL, 1), lambda i: (i, 0)),
            pl.BlockSpec((_RB_SEL, 1), lambda i: (i, 0)),
        ],
        out_shape=[
            jax.ShapeDtypeStruct((N, 1), jnp.float32),
            jax.ShapeDtypeStruct((N, 1), jnp.int32),
        ],
        scratch_shapes=[pltpu.VMEM((_RB_SEL, N), jnp.int32)],
        compiler_params=pltpu.CompilerParams(
            dimension_semantics=("parallel",)),
    )(s)


def _assemble_kernel(s_ref, adj_ref, vr_ref, cr_ref, vc_ref, cc_ref,
                     sym_ref, fin_ref, deg_ref):
    i = pl.program_id(0)
    s = s_ref[...]
    col = jax.lax.broadcasted_iota(jnp.int32, s.shape, 1)
    row = jax.lax.broadcasted_iota(jnp.int32, s.shape, 0) + i * _RB_ASM
    vr, cr = vr_ref[...], cr_ref[...]
    vc, cc = vc_ref[...], cc_ref[...]
    mrow = (s > vr) | ((s == vr) & (col <= cr))
    mcol = (s > vc) | ((s == vc) & (row <= cc))
    asym = (0.5 * s) * (mrow.astype(jnp.float32) + mcol.astype(jnp.float32))
    fin = asym + adj_ref[...]
    sym_ref[...] = asym
    fin_ref[...] = fin
    deg_ref[...] = jnp.sum(fin, axis=1, keepdims=True)


def _assemble(s, adj, v30, cut):
    return pl.pallas_call(
        _assemble_kernel,
        grid=(N // _RB_ASM,),
        in_specs=[
            pl.BlockSpec((_RB_ASM, N), lambda i: (i, 0)),
            pl.BlockSpec((_RB_ASM, N), lambda i: (i, 0)),
            pl.BlockSpec((_RB_ASM, 1), lambda i: (i, 0)),
            pl.BlockSpec((_RB_ASM, 1), lambda i: (i, 0)),
            pl.BlockSpec((1, N), lambda i: (0, 0)),
            pl.BlockSpec((1, N), lambda i: (0, 0)),
        ],
        out_specs=[
            pl.BlockSpec((_RB_ASM, N), lambda i: (i, 0)),
            pl.BlockSpec((_RB_ASM, N), lambda i: (i, 0)),
            pl.BlockSpec((_RB_ASM, 1), lambda i: (i, 0)),
        ],
        out_shape=[
            jax.ShapeDtypeStruct((N, N), jnp.float32),
            jax.ShapeDtypeStruct((N, N), jnp.float32),
            jax.ShapeDtypeStruct((N, 1), jnp.float32),
        ],
        compiler_params=pltpu.CompilerParams(
            dimension_semantics=("parallel",)),
    )(s, adj, v30, cut, v30.T, cut.T)


def _mm_scaled_kernel(a_ref, b_ref, scale_ref, bias_ref, o_ref, *, relu):
    r = jnp.dot(a_ref[...], b_ref[...], preferred_element_type=jnp.float32)
    r = r * scale_ref[...] + bias_ref[...]
    o_ref[...] = jnp.maximum(r, 0.0) if relu else r


def _mm_scaled(a, b, scale, bias, relu):
    d = b.shape[1]
    return pl.pallas_call(
        functools.partial(_mm_scaled_kernel, relu=relu),
        grid=(N // _RB_MM,),
        in_specs=[
            pl.BlockSpec((_RB_MM, N), lambda i: (i, 0)),
            pl.BlockSpec((N, d), lambda i: (0, 0)),
            pl.BlockSpec((_RB_MM, 1), lambda i: (i, 0)),
            pl.BlockSpec((1, d), lambda i: (0, 0)),
        ],
        out_specs=pl.BlockSpec((_RB_MM, d), lambda i: (i, 0)),
        out_shape=jax.ShapeDtypeStruct((N, d), jnp.float32),
        compiler_params=pltpu.CompilerParams(
            dimension_semantics=("parallel",)),
    )(a, b, scale, bias)


def kernel(input, Adj, W_enc1, b_enc1, W_enc2, b_enc2, metric_w,
           W_t1, b_t1, W_t2, b_t2):
    deg = jnp.sum(Adj, axis=1)
    dinv = jnp.where(deg > 0, 1.0 / jnp.sqrt(deg), 0.0)
    nA = Adj * dinv[:, None] * dinv[None, :]
    h = jax.nn.relu(nA @ (input @ W_enc1) + b_enc1)
    emb = nA @ (h @ W_enc2) + b_enc2
    S = jnp.zeros((N, N), dtype=jnp.float32)
    for p in range(P):
        hp = emb * metric_w[p]
        hp = hp / (jnp.linalg.norm(hp, axis=1, keepdims=True) + 1e-12)
        S = S + hp @ hp.T
    S = S / P

    v30, cut = _select(S)
    A_sym, A_final, deg_f = _assemble(S, Adj, v30, cut)
    dinv_f = jnp.where(deg_f > 0, 1.0 / jnp.sqrt(deg_f), 0.0)
    z1 = dinv_f * (input @ W_t1)
    x1 = _mm_scaled(A_final, z1, dinv_f, b_t1.reshape(1, -1), relu=True)
    z2 = dinv_f * (x1 @ W_t2)
    out = _mm_scaled(A_final, z2, dinv_f, b_t2.reshape(1, -1), relu=False)
    return (out, A_sym, A_final, emb)


# speedup vs baseline: 1.0456x; 1.0456x over previous
import functools
import jax
import jax.numpy as jnp
from jax.experimental import pallas as pl
from jax.experimental.pallas import tpu as pltpu

N = 10000
K = 30
P = 2

_RB_SEL = 80
_RB_ASM = 80
_RB_MM = 400

_INT_MIN = -(2 ** 31)


def _f32_key(s):
    b = jax.lax.bitcast_convert_type(s, jnp.int32)
    return jnp.where(b >= 0, b, (~b) ^ _INT_MIN)


def _key_f32(k):
    b = jnp.where(k >= 0, k, ~(k ^ _INT_MIN))
    return jax.lax.bitcast_convert_type(b, jnp.float32)


def _select_kernel(s_ref, ov_ref, oc_ref, key_ref):
    key_ref[...] = _f32_key(s_ref[...])
    key = key_ref[...]
    lo = jnp.min(key, axis=1, keepdims=True)
    hi = jnp.max(key, axis=1, keepdims=True) + 1

    def body(_, carry):
        def active(c):
            lo, hi = c
            mid = (lo & hi) + ((lo ^ hi) >> 1)
            cnt = jnp.sum((key >= mid).astype(jnp.float32), axis=1,
                          keepdims=True)
            pred = cnt >= K
            return jnp.where(pred, mid, lo), jnp.where(pred, hi, mid)

        lo, hi = carry
        return jax.lax.cond(jnp.any(hi > lo + 1), active, lambda c: c,
                            (lo, hi))

    lo, hi = jax.lax.fori_loop(0, 32, body, (lo, hi))
    k30 = lo

    m = K - jnp.sum((key > k30).astype(jnp.float32), axis=1, keepdims=True)
    col = jax.lax.broadcasted_iota(jnp.int32, key.shape, 1)
    colv = jnp.where(key == k30, col, N)

    def body2(_, carry):
        lo2, hi2 = carry
        mid = (lo2 + hi2) >> 1
        cnt = jnp.sum((colv <= mid).astype(jnp.float32), axis=1,
                      keepdims=True)
        pred = cnt >= m
        return jnp.where(pred, lo2, mid + 1), jnp.where(pred, mid, hi2)

    zero = jnp.zeros_like(k30)
    lo2, hi2 = jax.lax.fori_loop(0, 14, body2, (zero, zero + (N - 1)))

    ov_ref[...] = _key_f32(k30)
    oc_ref[...] = lo2


def _select(s):
    return pl.pallas_call(
        _select_kernel,
        grid=(N // _RB_SEL,),
        in_specs=[pl.BlockSpec((_RB_SEL, N), lambda i: (i, 0))],
        out_specs=[
            pl.BlockSpec((_RB_SEL, 1), lambda i: (i, 0)),
            pl.BlockSpec((_RB_SEL, 1), lambda i: (i, 0)),
        ],
        out_shape=[
            jax.ShapeDtypeStruct((N, 1), jnp.float32),
            jax.ShapeDtypeStruct((N, 1), jnp.int32),
        ],
        scratch_shapes=[pltpu.VMEM((_RB_SEL, N), jnp.int32)],
        compiler_params=pltpu.CompilerParams(
            dimension_semantics=("parallel",)),
    )(s)


def _assemble_kernel(s_ref, adj_ref, vr_ref, cr_ref, vc_ref, cc_ref,
                     sym_ref, fin_ref, deg_ref):
    i = pl.program_id(0)
    s = s_ref[...]
    col = jax.lax.broadcasted_iota(jnp.int32, s.shape, 1)
    row = jax.lax.broadcasted_iota(jnp.int32, s.shape, 0) + i * _RB_ASM
    vr, cr = vr_ref[...], cr_ref[...]
    vc, cc = vc_ref[...], cc_ref[...]
    mrow = (s > vr) | ((s == vr) & (col <= cr))
    mcol = (s > vc) | ((s == vc) & (row <= cc))
    asym = (0.5 * s) * (mrow.astype(jnp.float32) + mcol.astype(jnp.float32))
    fin = asym + adj_ref[...]
    sym_ref[...] = asym
    fin_ref[...] = fin
    deg_ref[...] = jnp.sum(fin, axis=1, keepdims=True)


def _assemble(s, adj, v30, cut):
    return pl.pallas_call(
        _assemble_kernel,
        grid=(N // _RB_ASM,),
        in_specs=[
            pl.BlockSpec((_RB_ASM, N), lambda i: (i, 0)),
            pl.BlockSpec((_RB_ASM, N), lambda i: (i, 0)),
            pl.BlockSpec((_RB_ASM, 1), lambda i: (i, 0)),
            pl.BlockSpec((_RB_ASM, 1), lambda i: (i, 0)),
            pl.BlockSpec((1, N), lambda i: (0, 0)),
            pl.BlockSpec((1, N), lambda i: (0, 0)),
        ],
        out_specs=[
            pl.BlockSpec((_RB_ASM, N), lambda i: (i, 0)),
            pl.BlockSpec((_RB_ASM, N), lambda i: (i, 0)),
            pl.BlockSpec((_RB_ASM, 1), lambda i: (i, 0)),
        ],
        out_shape=[
            jax.ShapeDtypeStruct((N, N), jnp.float32),
            jax.ShapeDtypeStruct((N, N), jnp.float32),
            jax.ShapeDtypeStruct((N, 1), jnp.float32),
        ],
        compiler_params=pltpu.CompilerParams(
            dimension_semantics=("parallel",)),
    )(s, adj, v30, cut, v30.T, cut.T)


def _mm_scaled_kernel(a_ref, b_ref, scale_ref, bias_ref, o_ref, *, relu):
    r = jnp.dot(a_ref[...], b_ref[...], preferred_element_type=jnp.float32)
    r = r * scale_ref[...] + bias_ref[...]
    o_ref[...] = jnp.maximum(r, 0.0) if relu else r


def _mm_scaled(a, b, scale, bias, relu):
    d = b.shape[1]
    return pl.pallas_call(
        functools.partial(_mm_scaled_kernel, relu=relu),
        grid=(N // _RB_MM,),
        in_specs=[
            pl.BlockSpec((_RB_MM, N), lambda i: (i, 0)),
            pl.BlockSpec((N, d), lambda i: (0, 0)),
            pl.BlockSpec((_RB_MM, 1), lambda i: (i, 0)),
            pl.BlockSpec((1, d), lambda i: (0, 0)),
        ],
        out_specs=pl.BlockSpec((_RB_MM, d), lambda i: (i, 0)),
        out_shape=jax.ShapeDtypeStruct((N, d), jnp.float32),
        compiler_params=pltpu.CompilerParams(
            dimension_semantics=("parallel",)),
    )(a, b, scale, bias)


def kernel(input, Adj, W_enc1, b_enc1, W_enc2, b_enc2, metric_w,
           W_t1, b_t1, W_t2, b_t2):
    deg = jnp.sum(Adj, axis=1)
    dinv = jnp.where(deg > 0, 1.0 / jnp.sqrt(deg), 0.0)
    nA = Adj * dinv[:, None] * dinv[None, :]
    h = jax.nn.relu(nA @ (input @ W_enc1) + b_enc1)
    emb = nA @ (h @ W_enc2) + b_enc2
    S = jnp.zeros((N, N), dtype=jnp.float32)
    for p in range(P):
        hp = emb * metric_w[p]
        hp = hp / (jnp.linalg.norm(hp, axis=1, keepdims=True) + 1e-12)
        S = S + hp @ hp.T
    S = S / P

    v30, cut = _select(S)
    A_sym, A_final, deg_f = _assemble(S, Adj, v30, cut)
    dinv_f = jnp.where(deg_f > 0, 1.0 / jnp.sqrt(deg_f), 0.0)
    z1 = dinv_f * (input @ W_t1)
    x1 = _mm_scaled(A_final, z1, dinv_f, b_t1.reshape(1, -1), relu=True)
    z2 = dinv_f * (x1 @ W_t2)
    out = _mm_scaled(A_final, z2, dinv_f, b_t2.reshape(1, -1), relu=False)
    return (out, A_sym, A_final, emb)


# speedup vs baseline: 1.1670x; 1.1162x over previous
import functools
import jax
import jax.numpy as jnp
from jax.experimental import pallas as pl
from jax.experimental.pallas import tpu as pltpu

N = 10000
K = 30
P = 2

_RB_SEL = 200
_RB_ASM = 80
_RB_MM = 400

_INT_MIN = -(2 ** 31)


def _f32_key(s):
    b = jax.lax.bitcast_convert_type(s, jnp.int32)
    return jnp.where(b >= 0, b, (~b) ^ _INT_MIN)


def _key_f32(k):
    b = jnp.where(k >= 0, k, ~(k ^ _INT_MIN))
    return jax.lax.bitcast_convert_type(b, jnp.float32)


def _select_kernel(s_ref, ov_ref, oc_ref, key_ref):
    key_ref[...] = _f32_key(s_ref[...])
    key = key_ref[...]
    lo = jnp.min(key, axis=1, keepdims=True)
    hi = jnp.max(key, axis=1, keepdims=True) + 1

    def body(_, carry):
        def active(c):
            lo, hi = c
            mid = (lo & hi) + ((lo ^ hi) >> 1)
            cnt = jnp.sum((key >= mid).astype(jnp.float32), axis=1,
                          keepdims=True)
            pred = cnt >= K
            return jnp.where(pred, mid, lo), jnp.where(pred, hi, mid)

        lo, hi = carry
        return jax.lax.cond(jnp.any(hi > lo + 1), active, lambda c: c,
                            (lo, hi))

    lo, hi = jax.lax.fori_loop(0, 32, body, (lo, hi))
    k30 = lo

    m = K - jnp.sum((key > k30).astype(jnp.float32), axis=1, keepdims=True)
    col = jax.lax.broadcasted_iota(jnp.int32, key.shape, 1)
    colv = jnp.where(key == k30, col, N)

    def body2(_, carry):
        lo2, hi2 = carry
        mid = (lo2 + hi2) >> 1
        cnt = jnp.sum((colv <= mid).astype(jnp.float32), axis=1,
                      keepdims=True)
        pred = cnt >= m
        return jnp.where(pred, lo2, mid + 1), jnp.where(pred, mid, hi2)

    zero = jnp.zeros_like(k30)
    lo2, hi2 = jax.lax.fori_loop(0, 14, body2, (zero, zero + (N - 1)))

    ov_ref[...] = _key_f32(k30)
    oc_ref[...] = lo2


def _select(s):
    return pl.pallas_call(
        _select_kernel,
        grid=(N // _RB_SEL,),
        in_specs=[pl.BlockSpec((_RB_SEL, N), lambda i: (i, 0))],
        out_specs=[
            pl.BlockSpec((_RB_SEL, 1), lambda i: (i, 0)),
            pl.BlockSpec((_RB_SEL, 1), lambda i: (i, 0)),
        ],
        out_shape=[
            jax.ShapeDtypeStruct((N, 1), jnp.float32),
            jax.ShapeDtypeStruct((N, 1), jnp.int32),
        ],
        scratch_shapes=[pltpu.VMEM((_RB_SEL, N), jnp.int32)],
        compiler_params=pltpu.CompilerParams(
            dimension_semantics=("parallel",)),
    )(s)


def _assemble_kernel(s_ref, adj_ref, vr_ref, cr_ref, vc_ref, cc_ref,
                     sym_ref, fin_ref, deg_ref):
    i = pl.program_id(0)
    s = s_ref[...]
    col = jax.lax.broadcasted_iota(jnp.int32, s.shape, 1)
    row = jax.lax.broadcasted_iota(jnp.int32, s.shape, 0) + i * _RB_ASM
    vr, cr = vr_ref[...], cr_ref[...]
    vc, cc = vc_ref[...], cc_ref[...]
    mrow = (s > vr) | ((s == vr) & (col <= cr))
    mcol = (s > vc) | ((s == vc) & (row <= cc))
    asym = (0.5 * s) * (mrow.astype(jnp.float32) + mcol.astype(jnp.float32))
    fin = asym + adj_ref[...]
    sym_ref[...] = asym
    fin_ref[...] = fin
    deg_ref[...] = jnp.sum(fin, axis=1, keepdims=True)


def _assemble(s, adj, v30, cut):
    return pl.pallas_call(
        _assemble_kernel,
        grid=(N // _RB_ASM,),
        in_specs=[
            pl.BlockSpec((_RB_ASM, N), lambda i: (i, 0)),
            pl.BlockSpec((_RB_ASM, N), lambda i: (i, 0)),
            pl.BlockSpec((_RB_ASM, 1), lambda i: (i, 0)),
            pl.BlockSpec((_RB_ASM, 1), lambda i: (i, 0)),
            pl.BlockSpec((1, N), lambda i: (0, 0)),
            pl.BlockSpec((1, N), lambda i: (0, 0)),
        ],
        out_specs=[
            pl.BlockSpec((_RB_ASM, N), lambda i: (i, 0)),
            pl.BlockSpec((_RB_ASM, N), lambda i: (i, 0)),
            pl.BlockSpec((_RB_ASM, 1), lambda i: (i, 0)),
        ],
        out_shape=[
            jax.ShapeDtypeStruct((N, N), jnp.float32),
            jax.ShapeDtypeStruct((N, N), jnp.float32),
            jax.ShapeDtypeStruct((N, 1), jnp.float32),
        ],
        compiler_params=pltpu.CompilerParams(
            dimension_semantics=("parallel",)),
    )(s, adj, v30, cut, v30.T, cut.T)


def _mm_scaled_kernel(a_ref, b_ref, scale_ref, bias_ref, o_ref, *, relu):
    r = jnp.dot(a_ref[...], b_ref[...], preferred_element_type=jnp.float32)
    r = r * scale_ref[...] + bias_ref[...]
    o_ref[...] = jnp.maximum(r, 0.0) if relu else r


def _mm_scaled(a, b, scale, bias, relu):
    d = b.shape[1]
    return pl.pallas_call(
        functools.partial(_mm_scaled_kernel, relu=relu),
        grid=(N // _RB_MM,),
        in_specs=[
            pl.BlockSpec((_RB_MM, N), lambda i: (i, 0)),
            pl.BlockSpec((N, d), lambda i: (0, 0)),
            pl.BlockSpec((_RB_MM, 1), lambda i: (i, 0)),
            pl.BlockSpec((1, d), lambda i: (0, 0)),
        ],
        out_specs=pl.BlockSpec((_RB_MM, d), lambda i: (i, 0)),
        out_shape=jax.ShapeDtypeStruct((N, d), jnp.float32),
        compiler_params=pltpu.CompilerParams(
            dimension_semantics=("parallel",)),
    )(a, b, scale, bias)


def kernel(input, Adj, W_enc1, b_enc1, W_enc2, b_enc2, metric_w,
           W_t1, b_t1, W_t2, b_t2):
    deg = jnp.sum(Adj, axis=1)
    dinv = jnp.where(deg > 0, 1.0 / jnp.sqrt(deg), 0.0)
    nA = Adj * dinv[:, None] * dinv[None, :]
    h = jax.nn.relu(nA @ (input @ W_enc1) + b_enc1)
    emb = nA @ (h @ W_enc2) + b_enc2
    S = jnp.zeros((N, N), dtype=jnp.float32)
    for p in range(P):
        hp = emb * metric_w[p]
        hp = hp / (jnp.linalg.norm(hp, axis=1, keepdims=True) + 1e-12)
        S = S + hp @ hp.T
    S = S / P

    v30, cut = _select(S)
    A_sym, A_final, deg_f = _assemble(S, Adj, v30, cut)
    dinv_f = jnp.where(deg_f > 0, 1.0 / jnp.sqrt(deg_f), 0.0)
    z1 = dinv_f * (input @ W_t1)
    x1 = _mm_scaled(A_final, z1, dinv_f, b_t1.reshape(1, -1), relu=True)
    z2 = dinv_f * (x1 @ W_t2)
    out = _mm_scaled(A_final, z2, dinv_f, b_t2.reshape(1, -1), relu=False)
    return (out, A_sym, A_final, emb)


# speedup vs baseline: 1.2786x; 1.0956x over previous
import functools
import jax
import jax.numpy as jnp
from jax.experimental import pallas as pl
from jax.experimental.pallas import tpu as pltpu

N = 10000
K = 30
P = 2

_RB_SEL = 200
_RB_ASM = 80
_RB_MM = 400

_INT_MIN = -(2 ** 31)


def _f32_key(s):
    b = jax.lax.bitcast_convert_type(s, jnp.int32)
    return jnp.where(b >= 0, b, (~b) ^ _INT_MIN)


def _key_f32(k):
    b = jnp.where(k >= 0, k, ~(k ^ _INT_MIN))
    return jax.lax.bitcast_convert_type(b, jnp.float32)


def _select_kernel(s_ref, ov_ref, oc_ref, key_ref):
    key_ref[...] = _f32_key(s_ref[...])
    key = key_ref[...]
    lo = jnp.min(key, axis=1, keepdims=True)
    hi = jnp.max(key, axis=1, keepdims=True) + 1

    def body(_, carry):
        def active(c):
            lo, hi = c
            mid = (lo & hi) + ((lo ^ hi) >> 1)
            cnt = jnp.sum((key >= mid).astype(jnp.float32), axis=1,
                          keepdims=True)
            pred = cnt >= K
            return jnp.where(pred, mid, lo), jnp.where(pred, hi, mid)

        lo, hi = carry
        return jax.lax.cond(jnp.any(hi > lo + 1), active, lambda c: c,
                            (lo, hi))

    lo, hi = jax.lax.fori_loop(0, 32, body, (lo, hi))
    k30 = lo

    m = K - jnp.sum((key > k30).astype(jnp.float32), axis=1, keepdims=True)
    col = jax.lax.broadcasted_iota(jnp.int32, key.shape, 1)
    eq = key == k30
    colv = jnp.where(eq, col, N)

    CH = 125
    chunk_of = jax.lax.broadcasted_iota(jnp.int32, (N, 128), 1)
    cmat = (jax.lax.broadcasted_iota(jnp.int32, (N, 128), 0) // CH
            == chunk_of).astype(jnp.float32)
    tcnt = jnp.dot(eq.astype(jnp.float32), cmat,
                   preferred_element_type=jnp.float32)
    tri = (jax.lax.broadcasted_iota(jnp.int32, (128, 128), 0)
           <= jax.lax.broadcasted_iota(jnp.int32, (128, 128), 1))
    cum = jnp.dot(tcnt, tri.astype(jnp.float32),
                  preferred_element_type=jnp.float32)
    below = cum < m
    ch = jnp.sum(below.astype(jnp.int32), axis=1, keepdims=True)
    mres = m - jnp.max(jnp.where(below, cum, 0.0), axis=1, keepdims=True)

    def body2(_, carry):
        lo2, hi2 = carry
        mid = (lo2 + hi2) >> 1
        cnt = jnp.sum((colv <= mid).astype(jnp.float32), axis=1,
                      keepdims=True)
        pred = cnt >= m
        return jnp.where(pred, lo2, mid + 1), jnp.where(pred, mid, hi2)

    del mres
    lo2, hi2 = jax.lax.fori_loop(0, 7, body2,
                                 (ch * CH, ch * CH + (CH - 1)))

    ov_ref[...] = _key_f32(k30)
    oc_ref[...] = lo2


def _select(s):
    return pl.pallas_call(
        _select_kernel,
        grid=(N // _RB_SEL,),
        in_specs=[pl.BlockSpec((_RB_SEL, N), lambda i: (i, 0))],
        out_specs=[
            pl.BlockSpec((_RB_SEL, 1), lambda i: (i, 0)),
            pl.BlockSpec((_RB_SEL, 1), lambda i: (i, 0)),
        ],
        out_shape=[
            jax.ShapeDtypeStruct((N, 1), jnp.float32),
            jax.ShapeDtypeStruct((N, 1), jnp.int32),
        ],
        scratch_shapes=[pltpu.VMEM((_RB_SEL, N), jnp.int32)],
        compiler_params=pltpu.CompilerParams(
            dimension_semantics=("parallel",)),
    )(s)


def _assemble_kernel(s_ref, adj_ref, vr_ref, cr_ref, vc_ref, cc_ref,
                     sym_ref, fin_ref, deg_ref):
    i = pl.program_id(0)
    s = s_ref[...]
    col = jax.lax.broadcasted_iota(jnp.int32, s.shape, 1)
    row = jax.lax.broadcasted_iota(jnp.int32, s.shape, 0) + i * _RB_ASM
    vr, cr = vr_ref[...], cr_ref[...]
    vc, cc = vc_ref[...], cc_ref[...]
    mrow = (s > vr) | ((s == vr) & (col <= cr))
    mcol = (s > vc) | ((s == vc) & (row <= cc))
    asym = (0.5 * s) * (mrow.astype(jnp.float32) + mcol.astype(jnp.float32))
    fin = asym + adj_ref[...]
    sym_ref[...] = asym
    fin_ref[...] = fin
    deg_ref[...] = jnp.sum(fin, axis=1, keepdims=True)


def _assemble(s, adj, v30, cut):
    return pl.pallas_call(
        _assemble_kernel,
        grid=(N // _RB_ASM,),
        in_specs=[
            pl.BlockSpec((_RB_ASM, N), lambda i: (i, 0)),
            pl.BlockSpec((_RB_ASM, N), lambda i: (i, 0)),
            pl.BlockSpec((_RB_ASM, 1), lambda i: (i, 0)),
            pl.BlockSpec((_RB_ASM, 1), lambda i: (i, 0)),
            pl.BlockSpec((1, N), lambda i: (0, 0)),
            pl.BlockSpec((1, N), lambda i: (0, 0)),
        ],
        out_specs=[
            pl.BlockSpec((_RB_ASM, N), lambda i: (i, 0)),
            pl.BlockSpec((_RB_ASM, N), lambda i: (i, 0)),
            pl.BlockSpec((_RB_ASM, 1), lambda i: (i, 0)),
        ],
        out_shape=[
            jax.ShapeDtypeStruct((N, N), jnp.float32),
            jax.ShapeDtypeStruct((N, N), jnp.float32),
            jax.ShapeDtypeStruct((N, 1), jnp.float32),
        ],
        compiler_params=pltpu.CompilerParams(
            dimension_semantics=("parallel",)),
    )(s, adj, v30, cut, v30.T, cut.T)


def _mm_scaled_kernel(a_ref, b_ref, scale_ref, bias_ref, o_ref, *, relu):
    r = jnp.dot(a_ref[...], b_ref[...], preferred_element_type=jnp.float32)
    r = r * scale_ref[...] + bias_ref[...]
    o_ref[...] = jnp.maximum(r, 0.0) if relu else r


def _mm_scaled(a, b, scale, bias, relu):
    d = b.shape[1]
    return pl.pallas_call(
        functools.partial(_mm_scaled_kernel, relu=relu),
        grid=(N // _RB_MM,),
        in_specs=[
            pl.BlockSpec((_RB_MM, N), lambda i: (i, 0)),
            pl.BlockSpec((N, d), lambda i: (0, 0)),
            pl.BlockSpec((_RB_MM, 1), lambda i: (i, 0)),
            pl.BlockSpec((1, d), lambda i: (0, 0)),
        ],
        out_specs=pl.BlockSpec((_RB_MM, d), lambda i: (i, 0)),
        out_shape=jax.ShapeDtypeStruct((N, d), jnp.float32),
        compiler_params=pltpu.CompilerParams(
            dimension_semantics=("parallel",)),
    )(a, b, scale, bias)


def kernel(input, Adj, W_enc1, b_enc1, W_enc2, b_enc2, metric_w,
           W_t1, b_t1, W_t2, b_t2):
    deg = jnp.sum(Adj, axis=1)
    dinv = jnp.where(deg > 0, 1.0 / jnp.sqrt(deg), 0.0)
    nA = Adj * dinv[:, None] * dinv[None, :]
    h = jax.nn.relu(nA @ (input @ W_enc1) + b_enc1)
    emb = nA @ (h @ W_enc2) + b_enc2
    S = jnp.zeros((N, N), dtype=jnp.float32)
    for p in range(P):
        hp = emb * metric_w[p]
        hp = hp / (jnp.linalg.norm(hp, axis=1, keepdims=True) + 1e-12)
        S = S + hp @ hp.T
    S = S / P

    v30, cut = _select(S)
    A_sym, A_final, deg_f = _assemble(S, Adj, v30, cut)
    dinv_f = jnp.where(deg_f > 0, 1.0 / jnp.sqrt(deg_f), 0.0)
    z1 = dinv_f * (input @ W_t1)
    x1 = _mm_scaled(A_final, z1, dinv_f, b_t1.reshape(1, -1), relu=True)
    z2 = dinv_f * (x1 @ W_t2)
    out = _mm_scaled(A_final, z2, dinv_f, b_t2.reshape(1, -1), relu=False)
    return (out, A_sym, A_final, emb)


# speedup vs baseline: 1.2794x; 1.0006x over previous
import functools
import jax
import jax.numpy as jnp
from jax.experimental import pallas as pl
from jax.experimental.pallas import tpu as pltpu

N = 10000
K = 30
P = 2

_RB_SEL = 200
_RB_ASM = 80
_RB_MM = 400

_INT_MIN = -(2 ** 31)


def _f32_key(s):
    b = jax.lax.bitcast_convert_type(s, jnp.int32)
    return jnp.where(b >= 0, b, (~b) ^ _INT_MIN)


def _key_f32(k):
    b = jnp.where(k >= 0, k, ~(k ^ _INT_MIN))
    return jax.lax.bitcast_convert_type(b, jnp.float32)


def _select_kernel(s_ref, ov_ref, oc_ref, key_ref):
    key_ref[...] = _f32_key(s_ref[...])
    key = key_ref[...]
    lo = jnp.min(key, axis=1, keepdims=True)
    hi = jnp.max(key, axis=1, keepdims=True) + 1

    def body(_, carry):
        def active(c):
            lo, hi = c
            mid = (lo & hi) + ((lo ^ hi) >> 1)
            cnt = jnp.sum((key >= mid).astype(jnp.float32), axis=1,
                          keepdims=True)
            pred = cnt >= K
            return jnp.where(pred, mid, lo), jnp.where(pred, hi, mid)

        lo, hi = carry
        return jax.lax.cond(jnp.any(hi > lo + 1), active, lambda c: c,
                            (lo, hi))

    lo, hi = jax.lax.fori_loop(0, 32, body, (lo, hi))
    k30 = lo

    m = K - jnp.sum((key > k30).astype(jnp.float32), axis=1, keepdims=True)
    col = jax.lax.broadcasted_iota(jnp.int32, key.shape, 1)
    eq = key == k30
    colv = jnp.where(eq, col, N)

    CH = 125
    chunk_of = jax.lax.broadcasted_iota(jnp.int32, (N, 128), 1)
    cmat = (jax.lax.broadcasted_iota(jnp.int32, (N, 128), 0) // CH
            == chunk_of).astype(jnp.float32)
    tcnt = jnp.dot(eq.astype(jnp.float32), cmat,
                   preferred_element_type=jnp.float32)
    tri = (jax.lax.broadcasted_iota(jnp.int32, (128, 128), 0)
           <= jax.lax.broadcasted_iota(jnp.int32, (128, 128), 1))
    cum = jnp.dot(tcnt, tri.astype(jnp.float32),
                  preferred_element_type=jnp.float32)
    ch = jnp.sum((cum < m).astype(jnp.int32), axis=1, keepdims=True)

    def body2(_, carry):
        lo2, hi2 = carry
        mid = (lo2 + hi2) >> 1
        cnt = jnp.sum((colv <= mid).astype(jnp.float32), axis=1,
                      keepdims=True)
        pred = cnt >= m
        return jnp.where(pred, lo2, mid + 1), jnp.where(pred, mid, hi2)

    lo2, hi2 = jax.lax.fori_loop(0, 7, body2,
                                 (ch * CH, ch * CH + (CH - 1)))

    ov_ref[...] = _key_f32(k30)
    oc_ref[...] = lo2


def _select(s):
    return pl.pallas_call(
        _select_kernel,
        grid=(N // _RB_SEL,),
        in_specs=[pl.BlockSpec((_RB_SEL, N), lambda i: (i, 0))],
        out_specs=[
            pl.BlockSpec((_RB_SEL, 1), lambda i: (i, 0)),
            pl.BlockSpec((_RB_SEL, 1), lambda i: (i, 0)),
        ],
        out_shape=[
            jax.ShapeDtypeStruct((N, 1), jnp.float32),
            jax.ShapeDtypeStruct((N, 1), jnp.int32),
        ],
        scratch_shapes=[pltpu.VMEM((_RB_SEL, N), jnp.int32)],
        compiler_params=pltpu.CompilerParams(
            dimension_semantics=("parallel",)),
    )(s)


def _assemble_kernel(s_ref, adj_ref, vr_ref, cr_ref, vc_ref, cc_ref,
                     sym_ref, fin_ref, deg_ref):
    i = pl.program_id(0)
    s = s_ref[...]
    col = jax.lax.broadcasted_iota(jnp.int32, s.shape, 1)
    row = jax.lax.broadcasted_iota(jnp.int32, s.shape, 0) + i * _RB_ASM
    vr, cr = vr_ref[...], cr_ref[...]
    vc, cc = vc_ref[...], cc_ref[...]
    mrow = (s > vr) | ((s == vr) & (col <= cr))
    mcol = (s > vc) | ((s == vc) & (row <= cc))
    asym = (0.5 * s) * (mrow.astype(jnp.float32) + mcol.astype(jnp.float32))
    fin = asym + adj_ref[...]
    sym_ref[...] = asym
    fin_ref[...] = fin
    deg_ref[...] = jnp.sum(fin, axis=1, keepdims=True)


def _assemble(s, adj, v30, cut):
    return pl.pallas_call(
        _assemble_kernel,
        grid=(N // _RB_ASM,),
        in_specs=[
            pl.BlockSpec((_RB_ASM, N), lambda i: (i, 0)),
            pl.BlockSpec((_RB_ASM, N), lambda i: (i, 0)),
            pl.BlockSpec((_RB_ASM, 1), lambda i: (i, 0)),
            pl.BlockSpec((_RB_ASM, 1), lambda i: (i, 0)),
            pl.BlockSpec((1, N), lambda i: (0, 0)),
            pl.BlockSpec((1, N), lambda i: (0, 0)),
        ],
        out_specs=[
            pl.BlockSpec((_RB_ASM, N), lambda i: (i, 0)),
            pl.BlockSpec((_RB_ASM, N), lambda i: (i, 0)),
            pl.BlockSpec((_RB_ASM, 1), lambda i: (i, 0)),
        ],
        out_shape=[
            jax.ShapeDtypeStruct((N, N), jnp.float32),
            jax.ShapeDtypeStruct((N, N), jnp.float32),
            jax.ShapeDtypeStruct((N, 1), jnp.float32),
        ],
        compiler_params=pltpu.CompilerParams(
            dimension_semantics=("parallel",)),
    )(s, adj, v30, cut, v30.T, cut.T)


def _mm_scaled_kernel(a_ref, b_ref, scale_ref, bias_ref, o_ref, *, relu):
    r = jnp.dot(a_ref[...], b_ref[...], preferred_element_type=jnp.float32)
    r = r * scale_ref[...] + bias_ref[...]
    o_ref[...] = jnp.maximum(r, 0.0) if relu else r


def _mm_scaled(a, b, scale, bias, relu):
    d = b.shape[1]
    return pl.pallas_call(
        functools.partial(_mm_scaled_kernel, relu=relu),
        grid=(N // _RB_MM,),
        in_specs=[
            pl.BlockSpec((_RB_MM, N), lambda i: (i, 0)),
            pl.BlockSpec((N, d), lambda i: (0, 0)),
            pl.BlockSpec((_RB_MM, 1), lambda i: (i, 0)),
            pl.BlockSpec((1, d), lambda i: (0, 0)),
        ],
        out_specs=pl.BlockSpec((_RB_MM, d), lambda i: (i, 0)),
        out_shape=jax.ShapeDtypeStruct((N, d), jnp.float32),
        compiler_params=pltpu.CompilerParams(
            dimension_semantics=("parallel",)),
    )(a, b, scale, bias)


def kernel(input, Adj, W_enc1, b_enc1, W_enc2, b_enc2, metric_w,
           W_t1, b_t1, W_t2, b_t2):
    deg = jnp.sum(Adj, axis=1)
    dinv = jnp.where(deg > 0, 1.0 / jnp.sqrt(deg), 0.0)
    nA = Adj * dinv[:, None] * dinv[None, :]
    h = jax.nn.relu(nA @ (input @ W_enc1) + b_enc1)
    emb = nA @ (h @ W_enc2) + b_enc2
    S = jnp.zeros((N, N), dtype=jnp.float32)
    for p in range(P):
        hp = emb * metric_w[p]
        hp = hp / (jnp.linalg.norm(hp, axis=1, keepdims=True) + 1e-12)
        S = S + hp @ hp.T
    S = S / P

    v30, cut = _select(S)
    A_sym, A_final, deg_f = _assemble(S, Adj, v30, cut)
    dinv_f = jnp.where(deg_f > 0, 1.0 / jnp.sqrt(deg_f), 0.0)
    z1 = dinv_f * (input @ W_t1)
    x1 = _mm_scaled(A_final, z1, dinv_f, b_t1.reshape(1, -1), relu=True)
    z2 = dinv_f * (x1 @ W_t2)
    out = _mm_scaled(A_final, z2, dinv_f, b_t2.reshape(1, -1), relu=False)
    return (out, A_sym, A_final, emb)
